# baseline (speedup 1.0000x reference)
import functools

import jax
import jax.numpy as jnp
from jax.experimental import pallas as pl

N = 10000
H1 = 4
C1 = 256
H3 = 6
NC = 121

_BLK = 1000


def _mm_kernel(x_ref, w_ref, o_ref):
    o_ref[...] = jnp.dot(x_ref[...], w_ref[...],
                         preferred_element_type=jnp.float32)


def _mm(x, w):
    n, k = x.shape
    m = w.shape[1]
    grid = n // _BLK
    return pl.pallas_call(
        _mm_kernel,
        grid=(grid,),
        in_specs=[
            pl.BlockSpec((_BLK, k), lambda i: (i, 0)),
            pl.BlockSpec((k, m), lambda i: (0, 0)),
        ],
        out_specs=pl.BlockSpec((_BLK, m), lambda i: (i, 0)),
        out_shape=jax.ShapeDtypeStruct((n, m), jnp.float32),
    )(x, w)


def _gat(x, ei, W, a_s, a_d, b, H, C, concat):
    n = x.shape[0]
    src, dst = ei[0], ei[1]
    h = _mm(x, W).reshape(n, H, C)
    alpha_src = jnp.sum(h * a_s[None], axis=-1)
    alpha_dst = jnp.sum(h * a_d[None], axis=-1)
    e = alpha_src[src] + alpha_dst[dst]
    e = jax.nn.leaky_relu(e, negative_slope=0.2)
    emax = jax.ops.segment_max(e, dst, num_segments=n)
    ee = jnp.exp(e - emax[dst])
    den = jax.ops.segment_sum(ee, dst, num_segments=n)
    alpha = ee / (den[dst] + 1e-16)
    out = jax.ops.segment_sum(h[src] * alpha[:, :, None], dst, num_segments=n)
    if concat:
        out = out.reshape(n, H * C)
    else:
        out = out.mean(axis=1)
    return out + b


def kernel(x, edge_index, W1, a_src1, a_dst1, b1, Wl1, bl1, W2, a_src2,
           a_dst2, b2, Wl2, bl2, W3, a_src3, a_dst3, b3, Wl3, bl3):
    n = x.shape[0]
    loops = jnp.arange(n, dtype=edge_index.dtype)
    ei = jnp.concatenate([edge_index, jnp.stack([loops, loops])], axis=1)
    h = jax.nn.elu(_gat(x, ei, W1, a_src1, a_dst1, b1, H1, C1, True)
                   + _mm(x, Wl1) + bl1)
    h = jax.nn.elu(_gat(h, ei, W2, a_src2, a_dst2, b2, H1, C1, True)
                   + _mm(h, Wl2) + bl2)
    out = (_gat(h, ei, W3, a_src3, a_dst3, b3, H3, NC, False)
           + _mm(h, Wl3) + bl3)
    return out


# speedup vs baseline: 4.9300x; 4.9300x over previous
import functools

import jax
import jax.numpy as jnp
from jax import lax
from jax.experimental import pallas as pl
from jax.experimental.pallas import tpu as pltpu
from jax.experimental.pallas import tpu_sc as plsc

N = 10000
H1 = 4
C1 = 256
H3 = 6
NC = 121
E = 320000
E_TOT = E + N

NWORK = 32
RPW = 313
NPAD = NWORK * RPW
CH = 64
LANES = 16
E_PAD = E_TOT + CH

_BLK = 1000


def _mm_kernel(x_ref, w_ref, o_ref):
    o_ref[...] = jnp.dot(x_ref[...], w_ref[...],
                         preferred_element_type=jnp.float32)


def _mm(x, w):
    n, k = x.shape
    m = w.shape[1]
    return pl.pallas_call(
        _mm_kernel,
        grid=(n // _BLK,),
        in_specs=[
            pl.BlockSpec((_BLK, k), lambda i: (i, 0)),
            pl.BlockSpec((k, m), lambda i: (0, 0)),
        ],
        out_specs=pl.BlockSpec((_BLK, m), lambda i: (i, 0)),
        out_shape=jax.ShapeDtypeStruct((n, m), jnp.float32),
    )(x, w)


@functools.lru_cache(maxsize=None)
def _make_sc_gat(H, C, head_sum):
    mesh = plsc.VectorSubcoreMesh(core_axis_name="c", subcore_axis_name="s")
    info = plsc.get_sparse_core_info()
    n_cores = info.num_cores
    CV = C // LANES
    if head_sum:
        out_t = jax.ShapeDtypeStruct((NWORK, RPW * C), jnp.float32)
    else:
        out_t = jax.ShapeDtypeStruct((H, NWORK, RPW * C), jnp.float32)

    @functools.partial(
        pl.kernel, mesh=mesh, out_type=out_t,
        compiler_params=pltpu.CompilerParams(use_tc_tiling_on_sc=False),
        scratch_types=[
            pltpu.VMEM((LANES,), jnp.int32),
            pltpu.VMEM((CH,), jnp.int32),
            pltpu.VMEM((CH + LANES,), jnp.int32),
            pltpu.VMEM((CH,), jnp.int32),
            pltpu.VMEM((CH, LANES), jnp.float32),
            pltpu.VMEM((CH, LANES), jnp.float32),
            pltpu.VMEM((CH, C), jnp.float32),
            pltpu.VMEM((RPW * LANES,), jnp.float32),
            pltpu.VMEM((RPW * LANES,), jnp.float32),
            pltpu.VMEM((RPW * C,), jnp.float32),
            pltpu.SemaphoreType.DMA,
            pltpu.SemaphoreType.DMA,
        ],
    )
    def sc_gat(srcs_h, dsts_h, tbl_h, as_h, ad_h, rows_h, out_h,
               tblrow, srcb, dstb, hidx, asrc, adst, rows, emax, den, acc,
               sem0, sem1):
        wid = lax.axis_index("s") * n_cores + lax.axis_index("c")
        pltpu.sync_copy(tbl_h.at[wid], tblrow)
        tv = tblrow[...]
        e_lo = tv[0]
        e_hi = tv[1]
        row_lo = tv[2]
        a_lo = (e_lo // 8) * 8
        nch = (e_hi - a_lo + CH - 1) // CH
        laneiota = lax.iota(jnp.int32, LANES)

        def init_body(i, _):
            emax[pl.ds(i * LANES, LANES)] = jnp.full((LANES,), -3e38,
                                                     jnp.float32)
            den[pl.ds(i * LANES, LANES)] = jnp.zeros((LANES,), jnp.float32)
            return 0
        lax.fori_loop(0, RPW, init_body, 0)

        def stage_chunk(ch):
            base = a_lo + ch * CH
            pltpu.sync_copy(srcs_h.at[pl.ds(base, CH)], srcb)
            pltpu.sync_copy(dsts_h.at[pl.ds(base, CH)],
                            dstb.at[pl.ds(0, CH)])
            cp0 = pltpu.async_copy(as_h.at[srcb], asrc, sem0)
            cp1 = pltpu.async_copy(ad_h.at[dstb.at[pl.ds(0, CH)]], adst, sem1)
            cp0.wait()
            cp1.wait()
            return base

        def dst_at(i):
            return dstb[pl.ds(i, LANES)][0]

        def edge_e(i):
            ev = asrc[i] + adst[i]
            return jnp.where(ev > 0, ev, jnp.float32(0.2) * ev)

        def p1_chunk(ch, _):
            base = stage_chunk(ch)

            def body(i, _):
                gidx = base + i

                @pl.when((gidx >= e_lo) & (gidx < e_hi))
                def _():
                    r = dst_at(i) - row_lo
                    ev = edge_e(i)
                    sl = pl.ds(r * LANES, LANES)
                    emax[sl] = jnp.maximum(emax[sl], ev)
                return 0
            lax.fori_loop(0, CH, body, 0)
            return 0
        lax.fori_loop(0, nch, p1_chunk, 0)

        def p2_chunk(ch, _):
            base = stage_chunk(ch)

            def body(i, _):
                gidx = base + i

                @pl.when((gidx >= e_lo) & (gidx < e_hi))
                def _():
                    r = dst_at(i) - row_lo
                    ev = edge_e(i)
                    sl = pl.ds(r * LANES, LANES)
                    den[sl] = den[sl] + jnp.exp(ev - emax[sl])
                return 0
            lax.fori_loop(0, CH, body, 0)
            return 0
        lax.fori_loop(0, nch, p2_chunk, 0)

        def zero_acc():
            def zbody(i, _):
                acc[pl.ds(i * LANES, LANES)] = jnp.zeros((LANES,),
                                                         jnp.float32)
                return 0
            lax.fori_loop(0, RPW * CV, zbody, 0)

        for h in range(H):
            if (not head_sum) or h == 0:
                zero_acc()

            def p3_chunk(ch, _, h=h):
                base = stage_chunk(ch)

                def hbody(j, _):
                    sl = pl.ds(j * LANES, LANES)
                    hidx[sl] = srcb[sl] * H + h
                    return 0
                lax.fori_loop(0, CH // LANES, hbody, 0)
                pltpu.async_copy(rows_h.at[hidx], rows, sem0).wait()

                def body(i, _):
                    gidx = base + i

                    @pl.when((gidx >= e_lo) & (gidx < e_hi))
                    def _():
                        r = dst_at(i) - row_lo
                        ev = edge_e(i)
                        sl = pl.ds(r * LANES, LANES)
                        al = jnp.exp(ev - emax[sl]) / (den[sl] +
                                                       jnp.float32(1e-16))
                        ab = jnp.full((LANES,), al[h], jnp.float32)
                        for j in range(CV):
                            asl = pl.ds(r * C + j * LANES, LANES)
                            acc[asl] = (acc[asl] +
                                        ab * rows[i, pl.ds(j * LANES, LANES)])
                    return 0
                lax.fori_loop(0, CH, body, 0)
                return 0
            lax.fori_loop(0, nch, p3_chunk, 0)
            if not head_sum:
                pltpu.sync_copy(acc, out_h.at[h, wid])
        if head_sum:
            pltpu.sync_copy(acc, out_h.at[wid])

    return sc_gat


def _pad16(a, used):
    return jnp.pad(a, ((0, 0), (0, LANES - used)))


def _gat_layer(hhat, srcs, dsts, tbl, a_s, a_d, b, H, C, head_sum):
    n = hhat.shape[0]
    hr = hhat.reshape(n, H, C)
    as_tbl = _pad16(jnp.sum(hr * a_s[None], axis=-1), H)
    ad_tbl = _pad16(jnp.sum(hr * a_d[None], axis=-1), H)
    rows = hhat.reshape(n * H, C)
    gat = _make_sc_gat(H, C, head_sum)(srcs, dsts, tbl, as_tbl, ad_tbl, rows)
    if head_sum:
        out = gat.reshape(NPAD, C)[:N] * (1.0 / H)
    else:
        out = gat.reshape(H, NWORK, RPW, C).transpose(1, 2, 0, 3)
        out = out.reshape(NPAD, H * C)[:N]
    return out + b


def kernel(x, edge_index, W1, a_src1, a_dst1, b1, Wl1, bl1, W2, a_src2,
           a_dst2, b2, Wl2, bl2, W3, a_src3, a_dst3, b3, Wl3, bl3):
    n = x.shape[0]
    loops = jnp.arange(n, dtype=edge_index.dtype)
    src = jnp.concatenate([edge_index[0], loops])
    dst = jnp.concatenate([edge_index[1], loops])
    order = jnp.argsort(dst)
    srcs = src[order]
    dsts = dst[order]
    bounds = jnp.searchsorted(
        dsts, jnp.arange(NWORK + 1, dtype=jnp.int32) * RPW).astype(jnp.int32)
    tbl = jnp.zeros((NWORK, LANES), jnp.int32)
    tbl = tbl.at[:, 0].set(bounds[:NWORK])
    tbl = tbl.at[:, 1].set(bounds[1:])
    tbl = tbl.at[:, 2].set(jnp.arange(NWORK, dtype=jnp.int32) * RPW)
    srcs = jnp.pad(srcs, (0, E_PAD - E_TOT))
    dsts = jnp.pad(dsts, (0, E_PAD - E_TOT))

    g1 = _gat_layer(_mm(x, W1), srcs, dsts, tbl, a_src1, a_dst1, b1,
                    H1, C1, False)
    h = jax.nn.elu(g1 + _mm(x, Wl1) + bl1)
    g2 = _gat_layer(_mm(h, W2), srcs, dsts, tbl, a_src2, a_dst2, b2,
                    H1, C1, False)
    h = jax.nn.elu(g2 + _mm(h, Wl2) + bl2)
    C3 = 128
    W3p = jnp.pad(W3.reshape(-1, H3, NC), ((0, 0), (0, 0), (0, C3 - NC)))
    W3p = W3p.reshape(-1, H3 * C3)
    a_s3 = jnp.pad(a_src3, ((0, 0), (0, C3 - NC)))
    a_d3 = jnp.pad(a_dst3, ((0, 0), (0, C3 - NC)))
    g3 = _gat_layer(_mm(h, W3p), srcs, dsts, tbl, a_s3, a_d3,
                    jnp.pad(b3, (0, C3 - NC)), H3, C3, True)
    out = g3[:, :NC] + _mm(h, Wl3) + bl3
    return out


# speedup vs baseline: 5.9750x; 1.2120x over previous
import functools

import jax
import jax.numpy as jnp
from jax import lax
from jax.experimental import pallas as pl
from jax.experimental.pallas import tpu as pltpu
from jax.experimental.pallas import tpu_sc as plsc

N = 10000
H1 = 4
C1 = 256
H3 = 6
NC = 121
E = 320000
E_TOT = E + N

NWORK = 32
RPW = 313
NPAD = NWORK * RPW
CH = 64
LANES = 16
E_PAD = E_TOT + CH

_BLK = 1000


def _mm_kernel(x_ref, w_ref, o_ref):
    o_ref[...] = jnp.dot(x_ref[...], w_ref[...],
                         preferred_element_type=jnp.float32)


def _mm(x, w):
    n, k = x.shape
    m = w.shape[1]
    return pl.pallas_call(
        _mm_kernel,
        grid=(n // _BLK,),
        in_specs=[
            pl.BlockSpec((_BLK, k), lambda i: (i, 0)),
            pl.BlockSpec((k, m), lambda i: (0, 0)),
        ],
        out_specs=pl.BlockSpec((_BLK, m), lambda i: (i, 0)),
        out_shape=jax.ShapeDtypeStruct((n, m), jnp.float32),
    )(x, w)


@functools.lru_cache(maxsize=None)
def _make_sc_gat(H, C, head_sum):
    mesh = plsc.VectorSubcoreMesh(core_axis_name="c", subcore_axis_name="s")
    info = plsc.get_sparse_core_info()
    n_cores = info.num_cores
    CV = C // LANES
    if head_sum:
        out_t = jax.ShapeDtypeStruct((NWORK, RPW * C), jnp.float32)
    else:
        out_t = jax.ShapeDtypeStruct((H, NWORK, RPW * C), jnp.float32)

    @functools.partial(
        pl.kernel, mesh=mesh, out_type=out_t,
        compiler_params=pltpu.CompilerParams(use_tc_tiling_on_sc=False),
        scratch_types=[
            pltpu.VMEM((LANES,), jnp.int32),
            pltpu.VMEM((CH,), jnp.int32),
            pltpu.VMEM((CH + LANES,), jnp.int32),
            pltpu.VMEM((CH,), jnp.int32),
            pltpu.VMEM((CH, LANES), jnp.float32),
            pltpu.VMEM((CH, LANES), jnp.float32),
            pltpu.VMEM((CH, C), jnp.float32),
            pltpu.VMEM((RPW * LANES,), jnp.float32),
            pltpu.VMEM((RPW * LANES,), jnp.float32),
            pltpu.VMEM((RPW * C,), jnp.float32),
            pltpu.SemaphoreType.DMA,
            pltpu.SemaphoreType.DMA,
        ],
    )
    def sc_gat(srcs_h, dsts_h, tbl_h, as_h, ad_h, rows_h,
               out_h, tblrow, srcb, dstb, hidx, asrc, adst, rows,
               emax, den, acc, sem0, sem1):
        wid = lax.axis_index("s") * n_cores + lax.axis_index("c")
        pltpu.sync_copy(tbl_h.at[wid], tblrow)
        tv = tblrow[...]
        e_lo = tv[0]
        e_hi = tv[1]
        row_lo = tv[2]
        a_lo = (e_lo // 8) * 8
        nch = (e_hi - a_lo + CH - 1) // CH
        laneiota = lax.iota(jnp.int32, LANES)

        def init_body(i, _):
            emax[pl.ds(i * LANES, LANES)] = jnp.full((LANES,), -3e38,
                                                     jnp.float32)
            den[pl.ds(i * LANES, LANES)] = jnp.zeros((LANES,), jnp.float32)
            return 0
        lax.fori_loop(0, RPW, init_body, 0)

        def stage_chunk(ch):
            base = a_lo + ch * CH
            pltpu.sync_copy(srcs_h.at[pl.ds(base, CH)], srcb)
            pltpu.sync_copy(dsts_h.at[pl.ds(base, CH)],
                            dstb.at[pl.ds(0, CH)])
            cp0 = pltpu.async_copy(as_h.at[srcb], asrc, sem0)
            cp1 = pltpu.async_copy(ad_h.at[dstb.at[pl.ds(0, CH)]], adst, sem1)
            cp0.wait()
            cp1.wait()
            return base

        def dst_at(i):
            return dstb[pl.ds(i, LANES)][0]

        def edge_e(i):
            ev = asrc[i] + adst[i]
            return jnp.where(ev > 0, ev, jnp.float32(0.2) * ev)

        def loop_bounds(base):
            lo = jnp.maximum(e_lo - base, 0)
            hi = jnp.minimum(e_hi - base, CH)
            return lo, hi

        def p12_chunk(ch, _):
            base = stage_chunk(ch)
            lo, hi = loop_bounds(base)

            def body(i, _):
                r = dst_at(i) - row_lo
                ev = edge_e(i)
                sl = pl.ds(r * LANES, LANES)
                m_old = emax[sl]
                m_new = jnp.maximum(m_old, ev)
                den[sl] = (den[sl] * jnp.exp(m_old - m_new) +
                           jnp.exp(ev - m_new))
                emax[sl] = m_new
                return 0
            lax.fori_loop(lo, hi, body, 0)
            return 0
        lax.fori_loop(0, nch, p12_chunk, 0)

        def zero_acc():
            def zbody(i, _):
                acc[pl.ds(i * LANES, LANES)] = jnp.zeros((LANES,),
                                                         jnp.float32)
                return 0
            lax.fori_loop(0, RPW * CV, zbody, 0)

        for h in range(H):
            if (not head_sum) or h == 0:
                zero_acc()

            def p3_chunk(ch, _, h=h):
                base = a_lo + ch * CH
                pltpu.sync_copy(srcs_h.at[pl.ds(base, CH)], srcb)
                pltpu.sync_copy(dsts_h.at[pl.ds(base, CH)],
                                dstb.at[pl.ds(0, CH)])
                lo, hi = loop_bounds(base)

                def hbody(j, _):
                    sl = pl.ds(j * LANES, LANES)
                    hidx[sl] = srcb[sl] * H + h
                    return 0
                lax.fori_loop(0, CH // LANES, hbody, 0)
                cpr = pltpu.async_copy(rows_h.at[hidx], rows, sem0)
                cp0 = pltpu.async_copy(as_h.at[srcb], asrc, sem1)
                cp1 = pltpu.async_copy(ad_h.at[dstb.at[pl.ds(0, CH)]],
                                       adst, sem1)
                cp0.wait()
                cp1.wait()
                cpr.wait()

                def body(i, _):
                    r = dst_at(i) - row_lo
                    ev = edge_e(i)
                    sl = pl.ds(r * LANES, LANES)
                    al = jnp.exp(ev - emax[sl]) / (den[sl] +
                                                   jnp.float32(1e-16))
                    ab = jnp.full((LANES,), al[h], jnp.float32)
                    for j in range(CV):
                        asl = pl.ds(r * C + j * LANES, LANES)
                        acc[asl] = (acc[asl] +
                                    ab * rows[i, pl.ds(j * LANES, LANES)])
                    return 0
                lax.fori_loop(lo, hi, body, 0)
                return 0
            lax.fori_loop(0, nch, p3_chunk, 0)
            if not head_sum:
                pltpu.sync_copy(acc, out_h.at[h, wid])
        if head_sum:
            pltpu.sync_copy(acc, out_h.at[wid])

    return sc_gat


def _pad16(a, used):
    return jnp.pad(a, ((0, 0), (0, LANES - used)))


def _gat_layer(hhat, srcs, dsts, tbl, a_s, a_d, b, H, C, head_sum):
    n = hhat.shape[0]
    hr = hhat.reshape(n, H, C)
    as_tbl = _pad16(jnp.sum(hr * a_s[None], axis=-1), H)
    ad_tbl = _pad16(jnp.sum(hr * a_d[None], axis=-1), H)
    rows = hhat.reshape(n * H, C)
    gat = _make_sc_gat(H, C, head_sum)(
        srcs, dsts, tbl, as_tbl, ad_tbl, rows)
    if head_sum:
        out = gat.reshape(NPAD, C)[:N] * (1.0 / H)
    else:
        out = gat.reshape(H, NWORK, RPW, C).transpose(1, 2, 0, 3)
        out = out.reshape(NPAD, H * C)[:N]
    return out + b


def kernel(x, edge_index, W1, a_src1, a_dst1, b1, Wl1, bl1, W2, a_src2,
           a_dst2, b2, Wl2, bl2, W3, a_src3, a_dst3, b3, Wl3, bl3):
    n = x.shape[0]
    loops = jnp.arange(n, dtype=edge_index.dtype)
    src = jnp.concatenate([edge_index[0], loops])
    dst = jnp.concatenate([edge_index[1], loops])
    order = jnp.argsort(dst)
    srcs = src[order]
    dsts = dst[order]
    bounds = jnp.searchsorted(
        dsts, jnp.arange(NWORK + 1, dtype=jnp.int32) * RPW).astype(jnp.int32)
    tbl = jnp.zeros((NWORK, LANES), jnp.int32)
    tbl = tbl.at[:, 0].set(bounds[:NWORK])
    tbl = tbl.at[:, 1].set(bounds[1:])
    tbl = tbl.at[:, 2].set(jnp.arange(NWORK, dtype=jnp.int32) * RPW)
    srcs = jnp.pad(srcs, (0, E_PAD - E_TOT))
    dsts = jnp.pad(dsts, (0, E_PAD - E_TOT))

    g1 = _gat_layer(_mm(x, W1), srcs, dsts, tbl, a_src1, a_dst1, b1,
                    H1, C1, False)
    h = jax.nn.elu(g1 + _mm(x, Wl1) + bl1)
    g2 = _gat_layer(_mm(h, W2), srcs, dsts, tbl, a_src2, a_dst2, b2,
                    H1, C1, False)
    h = jax.nn.elu(g2 + _mm(h, Wl2) + bl2)
    C3 = 128
    W3p = jnp.pad(W3.reshape(-1, H3, NC), ((0, 0), (0, 0), (0, C3 - NC)))
    W3p = W3p.reshape(-1, H3 * C3)
    a_s3 = jnp.pad(a_src3, ((0, 0), (0, C3 - NC)))
    a_d3 = jnp.pad(a_dst3, ((0, 0), (0, C3 - NC)))
    g3 = _gat_layer(_mm(h, W3p), srcs, dsts, tbl, a_s3, a_d3,
                    jnp.pad(b3, (0, C3 - NC)), H3, C3, True)
    out = g3[:, :NC] + _mm(h, Wl3) + bl3
    return out


# speedup vs baseline: 6.7704x; 1.1331x over previous
import functools

import jax
import jax.numpy as jnp
from jax import lax
from jax.experimental import pallas as pl
from jax.experimental.pallas import tpu as pltpu
from jax.experimental.pallas import tpu_sc as plsc

N = 10000
H1 = 4
C1 = 256
H3 = 6
NC = 121
E = 320000
E_TOT = E + N

NWORK = 32
RPW = 313
NPAD = NWORK * RPW
CH = 64
LANES = 16
E_PAD = E_TOT + 2 * CH

_BLK = 1000


def _mm_kernel(x_ref, w_ref, o_ref):
    o_ref[...] = jnp.dot(x_ref[...], w_ref[...],
                         preferred_element_type=jnp.float32)


def _mm(x, w):
    n, k = x.shape
    m = w.shape[1]
    return pl.pallas_call(
        _mm_kernel,
        grid=(n // _BLK,),
        in_specs=[
            pl.BlockSpec((_BLK, k), lambda i: (i, 0)),
            pl.BlockSpec((k, m), lambda i: (0, 0)),
        ],
        out_specs=pl.BlockSpec((_BLK, m), lambda i: (i, 0)),
        out_shape=jax.ShapeDtypeStruct((n, m), jnp.float32),
    )(x, w)


@functools.lru_cache(maxsize=None)
def _make_sc_gat(H, C, head_sum):
    mesh = plsc.VectorSubcoreMesh(core_axis_name="c", subcore_axis_name="s")
    info = plsc.get_sparse_core_info()
    n_cores = info.num_cores
    CV = C // LANES
    if head_sum:
        out_t = jax.ShapeDtypeStruct((NWORK, RPW * C), jnp.float32)
    else:
        out_t = jax.ShapeDtypeStruct((H, NWORK, RPW * C), jnp.float32)

    slot_scratch = [
        pltpu.VMEM((CH,), jnp.int32),
        pltpu.VMEM((CH + LANES,), jnp.int32),
        pltpu.VMEM((CH,), jnp.int32),
        pltpu.VMEM((CH, LANES), jnp.float32),
        pltpu.VMEM((CH, LANES), jnp.float32),
        pltpu.VMEM((CH, C), jnp.float32),
        pltpu.SemaphoreType.DMA,
        pltpu.SemaphoreType.DMA,
        pltpu.SemaphoreType.DMA,
    ]

    @functools.partial(
        pl.kernel, mesh=mesh, out_type=out_t,
        compiler_params=pltpu.CompilerParams(use_tc_tiling_on_sc=False),
        scratch_types=[
            pltpu.VMEM((LANES,), jnp.int32),
            pltpu.VMEM((RPW * LANES,), jnp.float32),
            pltpu.VMEM((RPW * LANES,), jnp.float32),
            pltpu.VMEM((RPW * C,), jnp.float32),
        ] + slot_scratch + slot_scratch,
    )
    def sc_gat(srcs_h, dsts_h, tbl_h, as_h, ad_h, rows_h, out_h,
               tblrow, emax, den, acc, *slots):
        A = slots[:9]
        B = slots[9:]
        wid = lax.axis_index("s") * n_cores + lax.axis_index("c")
        pltpu.sync_copy(tbl_h.at[wid], tblrow)
        tv = tblrow[...]
        e_lo = tv[0]
        e_hi = tv[1]
        row_lo = tv[2]
        a_lo = (e_lo // 8) * 8
        nch = (e_hi - a_lo + CH - 1) // CH
        npair = (nch + 1) // 2

        def init_body(i, _):
            emax[pl.ds(i * LANES, LANES)] = jnp.full((LANES,), -3e38,
                                                     jnp.float32)
            den[pl.ds(i * LANES, LANES)] = jnp.zeros((LANES,), jnp.float32)
            return 0
        lax.fori_loop(0, RPW, init_body, 0)

        def issue_lin(ch, S):
            srcb, dstb, _, _, _, _, semL, _, _ = S
            base = a_lo + ch * CH
            pltpu.async_copy(srcs_h.at[pl.ds(base, CH)], srcb, semL)
            pltpu.async_copy(dsts_h.at[pl.ds(base, CH)],
                             dstb.at[pl.ds(0, CH)], semL)

        def drain_lin(S):
            srcb, dstb, _, _, _, _, semL, _, _ = S
            pltpu.make_async_copy(srcs_h.at[pl.ds(0, CH)], srcb, semL).wait()
            pltpu.make_async_copy(dsts_h.at[pl.ds(0, CH)],
                                  dstb.at[pl.ds(0, CH)], semL).wait()

        def issue_gath(S, h):
            srcb, dstb, hidx, asrc, adst, rows, _, semG, semR = S
            pltpu.async_copy(as_h.at[srcb], asrc, semG)
            pltpu.async_copy(ad_h.at[dstb.at[pl.ds(0, CH)]], adst, semG)
            if h is not None:
                def hbody(j, _):
                    sl = pl.ds(j * LANES, LANES)
                    hidx[sl] = srcb[sl] * H + h
                    return 0
                lax.fori_loop(0, CH // LANES, hbody, 0)
                pltpu.async_copy(rows_h.at[hidx], rows, semR)

        def drain_gath(S, h):
            srcb, dstb, hidx, asrc, adst, rows, _, semG, semR = S
            pltpu.make_async_copy(as_h.at[srcb], asrc, semG).wait()
            pltpu.make_async_copy(ad_h.at[dstb.at[pl.ds(0, CH)]],
                                  adst, semG).wait()
            if h is not None:
                pltpu.make_async_copy(rows_h.at[hidx], rows, semR).wait()

        def loop_bounds(base):
            lo = jnp.maximum(e_lo - base, 0)
            hi = jnp.minimum(e_hi - base, CH)
            return lo, hi

        def dst_at(S, i):
            dstb = S[1]
            return dstb[pl.ds(i, LANES)][0]

        def edge_e(S, i):
            asrc, adst = S[3], S[4]
            ev = asrc[i] + adst[i]
            return jnp.where(ev > 0, ev, jnp.float32(0.2) * ev)

        def pipe_pass(compute_chunk, h):
            def pair_body(p, _):
                c0 = 2 * p
                c1 = c0 + 1

                issue_lin(c0, A)

                @pl.when(c1 < nch)
                def _():
                    issue_lin(c1, B)

                drain_lin(A)
                issue_gath(A, h)

                @pl.when(c1 < nch)
                def _():
                    drain_lin(B)
                    issue_gath(B, h)

                drain_gath(A, h)
                compute_chunk(c0, A)

                @pl.when(c1 < nch)
                def _():
                    drain_gath(B, h)
                    compute_chunk(c1, B)
                return 0
            lax.fori_loop(0, npair, pair_body, 0)

        def p12_compute(ch, S):
            base = a_lo + ch * CH
            lo, hi = loop_bounds(base)

            def body(i, _):
                r = dst_at(S, i) - row_lo
                ev = edge_e(S, i)
                sl = pl.ds(r * LANES, LANES)
                m_old = emax[sl]
                m_new = jnp.maximum(m_old, ev)
                den[sl] = (den[sl] * jnp.exp(m_old - m_new) +
                           jnp.exp(ev - m_new))
                emax[sl] = m_new
                return 0
            lax.fori_loop(lo, hi, body, 0)
        pipe_pass(p12_compute, None)

        def zero_acc():
            def zbody(i, _):
                acc[pl.ds(i * LANES, LANES)] = jnp.zeros((LANES,),
                                                         jnp.float32)
                return 0
            lax.fori_loop(0, RPW * CV, zbody, 0)

        for h in range(H):
            if (not head_sum) or h == 0:
                zero_acc()

            def p3_compute(ch, S, h=h):
                base = a_lo + ch * CH
                lo, hi = loop_bounds(base)
                rows = S[5]

                def body(i, _):
                    r = dst_at(S, i) - row_lo
                    ev = edge_e(S, i)
                    sl = pl.ds(r * LANES, LANES)
                    al = jnp.exp(ev - emax[sl]) / (den[sl] +
                                                   jnp.float32(1e-16))
                    ab = jnp.full((LANES,), al[h], jnp.float32)
                    for j in range(CV):
                        asl = pl.ds(r * C + j * LANES, LANES)
                        acc[asl] = (acc[asl] +
                                    ab * rows[i, pl.ds(j * LANES, LANES)])
                    return 0
                lax.fori_loop(lo, hi, body, 0)
            pipe_pass(p3_compute, h)
            if not head_sum:
                pltpu.sync_copy(acc, out_h.at[h, wid])
        if head_sum:
            pltpu.sync_copy(acc, out_h.at[wid])

    return sc_gat


def _pad16(a, used):
    return jnp.pad(a, ((0, 0), (0, LANES - used)))


def _gat_layer(hhat, srcs, dsts, tbl, a_s, a_d, b, H, C, head_sum):
    n = hhat.shape[0]
    hr = hhat.reshape(n, H, C)
    as_tbl = _pad16(jnp.sum(hr * a_s[None], axis=-1), H)
    ad_tbl = _pad16(jnp.sum(hr * a_d[None], axis=-1), H)
    rows = hhat.reshape(n * H, C)
    gat = _make_sc_gat(H, C, head_sum)(
        srcs, dsts, tbl, as_tbl, ad_tbl, rows)
    if head_sum:
        out = gat.reshape(NPAD, C)[:N] * (1.0 / H)
    else:
        out = gat.reshape(H, NWORK, RPW, C).transpose(1, 2, 0, 3)
        out = out.reshape(NPAD, H * C)[:N]
    return out + b


def kernel(x, edge_index, W1, a_src1, a_dst1, b1, Wl1, bl1, W2, a_src2,
           a_dst2, b2, Wl2, bl2, W3, a_src3, a_dst3, b3, Wl3, bl3):
    n = x.shape[0]
    loops = jnp.arange(n, dtype=edge_index.dtype)
    src = jnp.concatenate([edge_index[0], loops])
    dst = jnp.concatenate([edge_index[1], loops])
    order = jnp.argsort(dst)
    srcs = src[order]
    dsts = dst[order]
    bounds = jnp.searchsorted(
        dsts, jnp.arange(NWORK + 1, dtype=jnp.int32) * RPW).astype(jnp.int32)
    tbl = jnp.zeros((NWORK, LANES), jnp.int32)
    tbl = tbl.at[:, 0].set(bounds[:NWORK])
    tbl = tbl.at[:, 1].set(bounds[1:])
    tbl = tbl.at[:, 2].set(jnp.arange(NWORK, dtype=jnp.int32) * RPW)
    srcs = jnp.pad(srcs, (0, E_PAD - E_TOT))
    dsts = jnp.pad(dsts, (0, E_PAD - E_TOT))

    g1 = _gat_layer(_mm(x, W1), srcs, dsts, tbl, a_src1, a_dst1, b1,
                    H1, C1, False)
    h = jax.nn.elu(g1 + _mm(x, Wl1) + bl1)
    g2 = _gat_layer(_mm(h, W2), srcs, dsts, tbl, a_src2, a_dst2, b2,
                    H1, C1, False)
    h = jax.nn.elu(g2 + _mm(h, Wl2) + bl2)
    C3 = 128
    W3p = jnp.pad(W3.reshape(-1, H3, NC), ((0, 0), (0, 0), (0, C3 - NC)))
    W3p = W3p.reshape(-1, H3 * C3)
    a_s3 = jnp.pad(a_src3, ((0, 0), (0, C3 - NC)))
    a_d3 = jnp.pad(a_dst3, ((0, 0), (0, C3 - NC)))
    g3 = _gat_layer(_mm(h, W3p), srcs, dsts, tbl, a_s3, a_d3,
                    jnp.pad(b3, (0, C3 - NC)), H3, C3, True)
    out = g3[:, :NC] + _mm(h, Wl3) + bl3
    return out


# speedup vs baseline: 9.1317x; 1.3488x over previous
import functools

import jax
import jax.numpy as jnp
from jax import lax
from jax.experimental import pallas as pl
from jax.experimental.pallas import tpu as pltpu
from jax.experimental.pallas import tpu_sc as plsc

N = 10000
H1 = 4
C1 = 256
H3 = 6
NC = 121
E = 320000
E_TOT = E + N

NWORK = 32
NSUB = 8
SUBR = 40
RPW = NSUB * SUBR
NPAD = NWORK * RPW
CH = 32
LANES = 16
E_PAD = E_TOT + 2 * CH

_BLK = 1000


def _mm_kernel(x_ref, w_ref, o_ref):
    o_ref[...] = jnp.dot(x_ref[...], w_ref[...],
                         preferred_element_type=jnp.float32)


def _mm(x, w):
    n, k = x.shape
    m = w.shape[1]
    return pl.pallas_call(
        _mm_kernel,
        grid=(n // _BLK,),
        in_specs=[
            pl.BlockSpec((_BLK, k), lambda i: (i, 0)),
            pl.BlockSpec((k, m), lambda i: (0, 0)),
        ],
        out_specs=pl.BlockSpec((_BLK, m), lambda i: (i, 0)),
        out_shape=jax.ShapeDtypeStruct((n, m), jnp.float32),
    )(x, w)


@functools.lru_cache(maxsize=None)
def _make_sc_gat(H, C, head_sum):
    mesh = plsc.VectorSubcoreMesh(core_axis_name="c", subcore_axis_name="s")
    info = plsc.get_sparse_core_info()
    n_cores = info.num_cores
    D = H * C
    CV = C // LANES
    AC = C if head_sum else D
    out_t = jax.ShapeDtypeStruct((NWORK * NSUB, SUBR * AC), jnp.float32)

    slot_scratch = [
        pltpu.VMEM((CH,), jnp.int32),
        pltpu.VMEM((CH + LANES,), jnp.int32),
        pltpu.VMEM((CH, LANES), jnp.float32),
        pltpu.VMEM((CH, LANES), jnp.float32),
        pltpu.VMEM((CH, D), jnp.float32),
        pltpu.SemaphoreType.DMA,
        pltpu.SemaphoreType.DMA,
        pltpu.SemaphoreType.DMA,
    ]

    @functools.partial(
        pl.kernel, mesh=mesh, out_type=out_t,
        compiler_params=pltpu.CompilerParams(use_tc_tiling_on_sc=False),
        scratch_types=[
            pltpu.VMEM((LANES,), jnp.int32),
            pltpu.VMEM((RPW * LANES,), jnp.float32),
            pltpu.VMEM((RPW * LANES,), jnp.float32),
            pltpu.VMEM((SUBR * AC,), jnp.float32),
        ] + slot_scratch + slot_scratch,
    )
    def sc_gat(srcs_h, dsts_h, tbl_h, as_h, ad_h, rows_h, out_h,
               tblrow, emax, den, acc, *slots):
        A = slots[:8]
        B = slots[8:]
        wid = lax.axis_index("s") * n_cores + lax.axis_index("c")
        row_lo = wid * RPW
        pltpu.sync_copy(tbl_h.at[wid], tblrow)
        tv = tblrow[...]

        def init_body(i, _):
            emax[pl.ds(i * LANES, LANES)] = jnp.full((LANES,), -3e38,
                                                     jnp.float32)
            den[pl.ds(i * LANES, LANES)] = jnp.zeros((LANES,), jnp.float32)
            return 0
        lax.fori_loop(0, RPW, init_body, 0)

        def issue_lin(a_lo, ch, S):
            srcb, dstb = S[0], S[1]
            semL = S[5]
            base = a_lo + ch * CH
            pltpu.async_copy(srcs_h.at[pl.ds(base, CH)], srcb, semL)
            pltpu.async_copy(dsts_h.at[pl.ds(base, CH)],
                             dstb.at[pl.ds(0, CH)], semL)

        def drain_lin(S):
            srcb, dstb = S[0], S[1]
            semL = S[5]
            pltpu.make_async_copy(srcs_h.at[pl.ds(0, CH)], srcb, semL).wait()
            pltpu.make_async_copy(dsts_h.at[pl.ds(0, CH)],
                                  dstb.at[pl.ds(0, CH)], semL).wait()

        def issue_gath(S, with_rows):
            srcb, dstb, asrc, adst, rows = S[:5]
            semG, semR = S[6], S[7]
            pltpu.async_copy(as_h.at[srcb], asrc, semG)
            pltpu.async_copy(ad_h.at[dstb.at[pl.ds(0, CH)]], adst, semG)
            if with_rows:
                pltpu.async_copy(rows_h.at[srcb], rows, semR)

        def drain_gath(S, with_rows):
            srcb, dstb, asrc, adst, rows = S[:5]
            semG, semR = S[6], S[7]
            pltpu.make_async_copy(as_h.at[srcb], asrc, semG).wait()
            pltpu.make_async_copy(ad_h.at[dstb.at[pl.ds(0, CH)]],
                                  adst, semG).wait()
            if with_rows:
                pltpu.make_async_copy(rows_h.at[srcb], rows, semR).wait()

        def dst_at(S, i):
            dstb = S[1]
            return dstb[pl.ds(i, LANES)][0]

        def edge_e(S, i):
            asrc, adst = S[2], S[3]
            ev = asrc[i] + adst[i]
            return jnp.where(ev > 0, ev, jnp.float32(0.2) * ev)

        def pipe_pass(e_lo, e_hi, compute_chunk, with_rows):
            a_lo = (e_lo // 8) * 8
            nch = (e_hi - a_lo + CH - 1) // CH
            npair = (nch + 1) // 2

            def bounds(ch):
                base = a_lo + ch * CH
                lo = jnp.maximum(e_lo - base, 0)
                hi = jnp.minimum(e_hi - base, CH)
                return lo, hi

            def pair_body(p, _):
                c0 = 2 * p
                c1 = c0 + 1

                issue_lin(a_lo, c0, A)

                @pl.when(c1 < nch)
                def _():
                    issue_lin(a_lo, c1, B)

                drain_lin(A)
                issue_gath(A, with_rows)

                @pl.when(c1 < nch)
                def _():
                    drain_lin(B)
                    issue_gath(B, with_rows)

                drain_gath(A, with_rows)
                lo, hi = bounds(c0)
                compute_chunk(lo, hi, A)

                @pl.when(c1 < nch)
                def _():
                    drain_gath(B, with_rows)
                    lo1, hi1 = bounds(c1)
                    compute_chunk(lo1, hi1, B)
                return 0
            lax.fori_loop(0, npair, pair_body, 0)

        def p12_compute(lo, hi, S):
            def body(i, _):
                r = dst_at(S, i) - row_lo
                ev = edge_e(S, i)
                sl = pl.ds(r * LANES, LANES)
                m_old = emax[sl]
                m_new = jnp.maximum(m_old, ev)
                den[sl] = (den[sl] * jnp.exp(m_old - m_new) +
                           jnp.exp(ev - m_new))
                emax[sl] = m_new
                return 0
            lax.fori_loop(lo, hi, body, 0)
        pipe_pass(tv[0], tv[NSUB], p12_compute, False)

        for q in range(NSUB):
            def zbody(i, _):
                acc[pl.ds(i * LANES, LANES)] = jnp.zeros((LANES,),
                                                         jnp.float32)
                return 0
            lax.fori_loop(0, SUBR * AC // LANES, zbody, 0)

            sub_lo = row_lo + q * SUBR

            def p3_compute(lo, hi, S, sub_lo=sub_lo):
                rows = S[4]

                def body(i, _):
                    d = dst_at(S, i)
                    r = d - row_lo
                    rq = d - sub_lo
                    ev = edge_e(S, i)
                    sl = pl.ds(r * LANES, LANES)
                    al = jnp.exp(ev - emax[sl]) / (den[sl] +
                                                   jnp.float32(1e-16))
                    if head_sum:
                        for j in range(CV):
                            asl = pl.ds(rq * C + j * LANES, LANES)
                            v = acc[asl]
                            for h in range(H):
                                ab = jnp.full((LANES,), al[h], jnp.float32)
                                v = v + ab * rows[i, pl.ds(h * C + j * LANES,
                                                           LANES)]
                            acc[asl] = v
                    else:
                        for h in range(H):
                            ab = jnp.full((LANES,), al[h], jnp.float32)
                            for j in range(CV):
                                o = h * C + j * LANES
                                asl = pl.ds(rq * D + o, LANES)
                                acc[asl] = (acc[asl] +
                                            ab * rows[i, pl.ds(o, LANES)])
                    return 0
                lax.fori_loop(lo, hi, body, 0)
            pipe_pass(tv[q], tv[q + 1], p3_compute, True)
            pltpu.sync_copy(acc, out_h.at[wid * NSUB + q])

    return sc_gat


def _pad16(a, used):
    return jnp.pad(a, ((0, 0), (0, LANES - used)))


def _gat_layer(hhat, srcs, dsts, tbl, a_s, a_d, b, H, C, head_sum):
    n = hhat.shape[0]
    hr = hhat.reshape(n, H, C)
    as_tbl = _pad16(jnp.sum(hr * a_s[None], axis=-1), H)
    ad_tbl = _pad16(jnp.sum(hr * a_d[None], axis=-1), H)
    gat = _make_sc_gat(H, C, head_sum)(
        srcs, dsts, tbl, as_tbl, ad_tbl, hhat)
    if head_sum:
        out = gat.reshape(NPAD, C)[:N] * (1.0 / H)
    else:
        out = gat.reshape(NPAD, H * C)[:N]
    return out + b


def kernel(x, edge_index, W1, a_src1, a_dst1, b1, Wl1, bl1, W2, a_src2,
           a_dst2, b2, Wl2, bl2, W3, a_src3, a_dst3, b3, Wl3, bl3):
    n = x.shape[0]
    loops = jnp.arange(n, dtype=edge_index.dtype)
    src = jnp.concatenate([edge_index[0], loops])
    dst = jnp.concatenate([edge_index[1], loops])
    order = jnp.argsort(dst)
    srcs = src[order]
    dsts = dst[order]
    bounds = jnp.searchsorted(
        dsts,
        jnp.arange(NWORK * NSUB + 1, dtype=jnp.int32) * SUBR,
    ).astype(jnp.int32)
    bidx = (jnp.arange(NWORK, dtype=jnp.int32)[:, None] * NSUB +
            jnp.arange(NSUB + 1, dtype=jnp.int32)[None, :])
    tbl = jnp.zeros((NWORK, LANES), jnp.int32)
    tbl = tbl.at[:, :NSUB + 1].set(bounds[bidx])
    srcs = jnp.pad(srcs, (0, E_PAD - E_TOT))
    dsts = jnp.pad(dsts, (0, E_PAD - E_TOT))

    g1 = _gat_layer(_mm(x, W1), srcs, dsts, tbl, a_src1, a_dst1, b1,
                    H1, C1, False)
    h = jax.nn.elu(g1 + _mm(x, Wl1) + bl1)
    g2 = _gat_layer(_mm(h, W2), srcs, dsts, tbl, a_src2, a_dst2, b2,
                    H1, C1, False)
    h = jax.nn.elu(g2 + _mm(h, Wl2) + bl2)
    C3 = 128
    W3p = jnp.pad(W3.reshape(-1, H3, NC), ((0, 0), (0, 0), (0, C3 - NC)))
    W3p = W3p.reshape(-1, H3 * C3)
    a_s3 = jnp.pad(a_src3, ((0, 0), (0, C3 - NC)))
    a_d3 = jnp.pad(a_dst3, ((0, 0), (0, C3 - NC)))
    g3 = _gat_layer(_mm(h, W3p), srcs, dsts, tbl, a_s3, a_d3,
                    jnp.pad(b3, (0, C3 - NC)), H3, C3, True)
    out = g3[:, :NC] + _mm(h, Wl3) + bl3
    return out


# speedup vs baseline: 9.2163x; 1.0093x over previous
import functools

import jax
import jax.numpy as jnp
from jax import lax
from jax.experimental import pallas as pl
from jax.experimental.pallas import tpu as pltpu
from jax.experimental.pallas import tpu_sc as plsc

N = 10000
H1 = 4
C1 = 256
H3 = 6
NC = 121
E = 320000
E_TOT = E + N

NWORK = 32
NSUB = 8
SUBR = 40
RPW = NSUB * SUBR
NPAD = NWORK * RPW
CHMAX = 64
LANES = 16
E_PAD = E_TOT + 2 * CHMAX

_BLK = 1000


def _mm_kernel(x_ref, w_ref, o_ref):
    o_ref[...] = jnp.dot(x_ref[...], w_ref[...],
                         preferred_element_type=jnp.float32)


def _mm(x, w):
    n, k = x.shape
    m = w.shape[1]
    return pl.pallas_call(
        _mm_kernel,
        grid=(n // _BLK,),
        in_specs=[
            pl.BlockSpec((_BLK, k), lambda i: (i, 0)),
            pl.BlockSpec((k, m), lambda i: (0, 0)),
        ],
        out_specs=pl.BlockSpec((_BLK, m), lambda i: (i, 0)),
        out_shape=jax.ShapeDtypeStruct((n, m), jnp.float32),
    )(x, w)


@functools.lru_cache(maxsize=None)
def _make_sc_gat(H, C, head_sum):
    mesh = plsc.VectorSubcoreMesh(core_axis_name="c", subcore_axis_name="s")
    info = plsc.get_sparse_core_info()
    n_cores = info.num_cores
    D = H * C
    CV = C // LANES
    AC = C if head_sum else D
    CH = 32 if D > 768 else 64
    out_t = jax.ShapeDtypeStruct((NWORK * NSUB, SUBR * AC), jnp.float32)

    slot_scratch = [
        pltpu.VMEM((CH,), jnp.int32),
        pltpu.VMEM((CH + LANES,), jnp.int32),
        pltpu.VMEM((CH, LANES), jnp.float32),
        pltpu.VMEM((CH, LANES), jnp.float32),
        pltpu.VMEM((CH, D), jnp.float32),
        pltpu.SemaphoreType.DMA,
        pltpu.SemaphoreType.DMA,
        pltpu.SemaphoreType.DMA,
    ]

    @functools.partial(
        pl.kernel, mesh=mesh, out_type=out_t,
        compiler_params=pltpu.CompilerParams(use_tc_tiling_on_sc=False),
        scratch_types=[
            pltpu.VMEM((LANES,), jnp.int32),
            pltpu.VMEM((RPW * LANES,), jnp.float32),
            pltpu.VMEM((RPW * LANES,), jnp.float32),
            pltpu.VMEM((SUBR * AC,), jnp.float32),
        ] + slot_scratch + slot_scratch,
    )
    def sc_gat(srcs_h, dsts_h, tbl_h, as_h, ad_h, rows_h, out_h,
               tblrow, emax, den, acc, *slots):
        A = slots[:8]
        B = slots[8:]
        wid = lax.axis_index("s") * n_cores + lax.axis_index("c")
        row_lo = wid * RPW
        pltpu.sync_copy(tbl_h.at[wid], tblrow)
        tv = tblrow[...]

        def init_body(i, _):
            emax[pl.ds(i * LANES, LANES)] = jnp.full((LANES,), -3e38,
                                                     jnp.float32)
            den[pl.ds(i * LANES, LANES)] = jnp.zeros((LANES,), jnp.float32)
            return 0
        lax.fori_loop(0, RPW, init_body, 0)

        def issue_lin(a_lo, ch, S):
            srcb, dstb = S[0], S[1]
            semL = S[5]
            base = a_lo + ch * CH
            pltpu.async_copy(srcs_h.at[pl.ds(base, CH)], srcb, semL)
            pltpu.async_copy(dsts_h.at[pl.ds(base, CH)],
                             dstb.at[pl.ds(0, CH)], semL)

        def drain_lin(S):
            srcb, dstb = S[0], S[1]
            semL = S[5]
            pltpu.make_async_copy(srcs_h.at[pl.ds(0, CH)], srcb, semL).wait()
            pltpu.make_async_copy(dsts_h.at[pl.ds(0, CH)],
                                  dstb.at[pl.ds(0, CH)], semL).wait()

        def issue_gath(S, with_rows):
            srcb, dstb, asrc, adst, rows = S[:5]
            semG, semR = S[6], S[7]
            pltpu.async_copy(as_h.at[srcb], asrc, semG)
            pltpu.async_copy(ad_h.at[dstb.at[pl.ds(0, CH)]], adst, semG)
            if with_rows:
                pltpu.async_copy(rows_h.at[srcb], rows, semR)

        def drain_gath(S, with_rows):
            srcb, dstb, asrc, adst, rows = S[:5]
            semG, semR = S[6], S[7]
            pltpu.make_async_copy(as_h.at[srcb], asrc, semG).wait()
            pltpu.make_async_copy(ad_h.at[dstb.at[pl.ds(0, CH)]],
                                  adst, semG).wait()
            if with_rows:
                pltpu.make_async_copy(rows_h.at[srcb], rows, semR).wait()

        def dst_at(S, i):
            dstb = S[1]
            return dstb[pl.ds(i, LANES)][0]

        def edge_e(S, i):
            asrc, adst = S[2], S[3]
            ev = asrc[i] + adst[i]
            return jnp.where(ev > 0, ev, jnp.float32(0.2) * ev)

        def pipe_pass(e_lo, e_hi, compute_chunk, with_rows):
            a_lo = (e_lo // 8) * 8
            nch = (e_hi - a_lo + CH - 1) // CH
            npair = (nch + 1) // 2

            def bounds(ch):
                base = a_lo + ch * CH
                lo = jnp.maximum(e_lo - base, 0)
                hi = jnp.minimum(e_hi - base, CH)
                return lo, hi

            def pair_body(p, _):
                c0 = 2 * p
                c1 = c0 + 1

                issue_lin(a_lo, c0, A)

                @pl.when(c1 < nch)
                def _():
                    issue_lin(a_lo, c1, B)

                drain_lin(A)
                issue_gath(A, with_rows)

                @pl.when(c1 < nch)
                def _():
                    drain_lin(B)
                    issue_gath(B, with_rows)

                drain_gath(A, with_rows)
                lo, hi = bounds(c0)
                compute_chunk(lo, hi, A)

                @pl.when(c1 < nch)
                def _():
                    drain_gath(B, with_rows)
                    lo1, hi1 = bounds(c1)
                    compute_chunk(lo1, hi1, B)
                return 0
            lax.fori_loop(0, npair, pair_body, 0)

        def p12_compute(lo, hi, S):
            def body(i, _):
                r = dst_at(S, i) - row_lo
                ev = edge_e(S, i)
                sl = pl.ds(r * LANES, LANES)
                m_old = emax[sl]
                m_new = jnp.maximum(m_old, ev)
                den[sl] = (den[sl] * jnp.exp(m_old - m_new) +
                           jnp.exp(ev - m_new))
                emax[sl] = m_new
                return 0
            lax.fori_loop(lo, hi, body, 0)
        pipe_pass(tv[0], tv[NSUB], p12_compute, False)

        for q in range(NSUB):
            def zbody(i, _):
                acc[pl.ds(i * LANES, LANES)] = jnp.zeros((LANES,),
                                                         jnp.float32)
                return 0
            lax.fori_loop(0, SUBR * AC // LANES, zbody, 0)

            sub_lo = row_lo + q * SUBR

            def p3_compute(lo, hi, S, sub_lo=sub_lo):
                rows = S[4]

                def body(i, _):
                    d = dst_at(S, i)
                    r = d - row_lo
                    rq = d - sub_lo
                    ev = edge_e(S, i)
                    sl = pl.ds(r * LANES, LANES)
                    al = jnp.exp(ev - emax[sl]) / (den[sl] +
                                                   jnp.float32(1e-16))
                    abs_ = [_splat(al, h) for h in range(H)]
                    if head_sum:
                        for j in range(CV):
                            asl = pl.ds(rq * C + j * LANES, LANES)
                            v = acc[asl]
                            for h in range(H):
                                v = (v + abs_[h] *
                                     rows[i, pl.ds(h * C + j * LANES,
                                                   LANES)])
                            acc[asl] = v
                    else:
                        for h in range(H):
                            for j in range(CV):
                                o = h * C + j * LANES
                                asl = pl.ds(rq * D + o, LANES)
                                acc[asl] = (acc[asl] +
                                            abs_[h] * rows[i, pl.ds(o,
                                                                    LANES)])
                    return 0
                lax.fori_loop(lo, hi, body, 0)
            pipe_pass(tv[q], tv[q + 1], p3_compute, True)
            pltpu.sync_copy(acc, out_h.at[wid * NSUB + q])

    return sc_gat


_GDN = lax.GatherDimensionNumbers(offset_dims=(),
                                  collapsed_slice_dims=(0,),
                                  start_index_map=(0,))


def _splat(v, h):
    idx = jnp.full((LANES, 1), h, jnp.int32)
    return lax.gather(v, idx, dimension_numbers=_GDN, slice_sizes=(1,),
                      mode=lax.GatherScatterMode.PROMISE_IN_BOUNDS)


def _pad16(a, used):
    return jnp.pad(a, ((0, 0), (0, LANES - used)))


def _gat_layer(hhat, srcs, dsts, tbl, a_s, a_d, b, H, C, head_sum):
    n = hhat.shape[0]
    hr = hhat.reshape(n, H, C)
    as_tbl = _pad16(jnp.sum(hr * a_s[None], axis=-1), H)
    ad_tbl = _pad16(jnp.sum(hr * a_d[None], axis=-1), H)
    gat = _make_sc_gat(H, C, head_sum)(
        srcs, dsts, tbl, as_tbl, ad_tbl, hhat)
    if head_sum:
        out = gat.reshape(NPAD, C)[:N] * (1.0 / H)
    else:
        out = gat.reshape(NPAD, H * C)[:N]
    return out + b


def kernel(x, edge_index, W1, a_src1, a_dst1, b1, Wl1, bl1, W2, a_src2,
           a_dst2, b2, Wl2, bl2, W3, a_src3, a_dst3, b3, Wl3, bl3):
    n = x.shape[0]
    loops = jnp.arange(n, dtype=edge_index.dtype)
    src = jnp.concatenate([edge_index[0], loops])
    dst = jnp.concatenate([edge_index[1], loops])
    order = jnp.argsort(dst)
    srcs = src[order]
    dsts = dst[order]
    bounds = jnp.searchsorted(
        dsts,
        jnp.arange(NWORK * NSUB + 1, dtype=jnp.int32) * SUBR,
    ).astype(jnp.int32)
    bidx = (jnp.arange(NWORK, dtype=jnp.int32)[:, None] * NSUB +
            jnp.arange(NSUB + 1, dtype=jnp.int32)[None, :])
    tbl = jnp.zeros((NWORK, LANES), jnp.int32)
    tbl = tbl.at[:, :NSUB + 1].set(bounds[bidx])
    srcs = jnp.pad(srcs, (0, E_PAD - E_TOT))
    dsts = jnp.pad(dsts, (0, E_PAD - E_TOT))

    g1 = _gat_layer(_mm(x, W1), srcs, dsts, tbl, a_src1, a_dst1, b1,
                    H1, C1, False)
    h = jax.nn.elu(g1 + _mm(x, Wl1) + bl1)
    g2 = _gat_layer(_mm(h, W2), srcs, dsts, tbl, a_src2, a_dst2, b2,
                    H1, C1, False)
    h = jax.nn.elu(g2 + _mm(h, Wl2) + bl2)
    C3 = 128
    W3p = jnp.pad(W3.reshape(-1, H3, NC), ((0, 0), (0, 0), (0, C3 - NC)))
    W3p = W3p.reshape(-1, H3 * C3)
    a_s3 = jnp.pad(a_src3, ((0, 0), (0, C3 - NC)))
    a_d3 = jnp.pad(a_dst3, ((0, 0), (0, C3 - NC)))
    g3 = _gat_layer(_mm(h, W3p), srcs, dsts, tbl, a_s3, a_d3,
                    jnp.pad(b3, (0, C3 - NC)), H3, C3, True)
    out = g3[:, :NC] + _mm(h, Wl3) + bl3
    return out
